# baseline (speedup 1.0000x reference)
import functools

import jax
import jax.numpy as jnp
from jax import lax
from jax.experimental import pallas as pl
from jax.experimental.pallas import tpu as pltpu
from jax.experimental.pallas import tpu_sc as plsc

NUM_CLASSES = 21
TOP_K = 200
CONF_THRESH = 0.01
NMS_THRESH = 0.45
OBJ_THRESH = 0.01
V0, V1 = 0.1, 0.2
B, P = 8, 16320

L = 16
ROWS = B * (NUM_CLASSES - 1)
NC, NS = 2, 16
NW = NC * NS
RPW = ROWS // NW
SELW = 256
NBV = 13
PP = 16384
NCHUNK = 4
CH = PP // NCHUNK
OW = 5 * SELW


def _decode_body(al_ref, ol_ref, pr_ref, cf_ref, ao_ref, out_ref, mc_ref):
    al = al_ref[0]
    ol = ol_ref[0]
    pr = pr_ref[...]
    pcx, pcy, pw, ph = pr[0:1], pr[1:2], pr[2:3], pr[3:4]
    dcx = pcx + al[0:1] * V0 * pw
    dcy = pcy + al[1:2] * V0 * ph
    dw = pw * jnp.exp(al[2:3] * V1)
    dh = ph * jnp.exp(al[3:4] * V1)
    x1 = dcx - dw / 2.0
    y1 = dcy - dh / 2.0
    x2 = dcx + dw / 2.0
    y2 = dcy + dh / 2.0
    dcx = (x2 + x1) / 2.0
    dcy = (y2 + y1) / 2.0
    dw = x2 - x1
    dh = y2 - y1
    bcx = dcx + ol[0:1] * V0 * dw
    bcy = dcy + ol[1:2] * V0 * dh
    bw = dw * jnp.exp(ol[2:3] * V1)
    bh = dh * jnp.exp(ol[3:4] * V1)
    zpad4 = jnp.zeros((4, PP - P), jnp.float32)
    zpad21 = jnp.zeros((NUM_CLASSES, PP - P), jnp.float32)
    box4 = jnp.concatenate(
        [bcx - bw / 2.0, bcy - bh / 2.0, bcx + bw / 2.0, bcy + bh / 2.0], axis=0)
    out_ref[0] = jnp.concatenate([box4, zpad4], axis=1)
    mc = jnp.where(ao_ref[0] > OBJ_THRESH, cf_ref[0], 0.0)
    mc_ref[0] = jnp.concatenate([mc, zpad21], axis=1)


def _decode_boxes(arm_loc, odm_loc, priors, conf_t, armobj):
    al_t = jnp.transpose(arm_loc, (0, 2, 1))
    ol_t = jnp.transpose(odm_loc, (0, 2, 1))
    pr_t = jnp.transpose(priors, (1, 0))
    boxes_t, mconf = pl.pallas_call(
        _decode_body,
        grid=(B,),
        in_specs=[
            pl.BlockSpec((1, 4, P), lambda b: (b, 0, 0)),
            pl.BlockSpec((1, 4, P), lambda b: (b, 0, 0)),
            pl.BlockSpec((4, P), lambda b: (0, 0)),
            pl.BlockSpec((1, NUM_CLASSES, P), lambda b: (b, 0, 0)),
            pl.BlockSpec((1, 1, P), lambda b: (b, 0, 0)),
        ],
        out_specs=[
            pl.BlockSpec((1, 4, PP), lambda b: (b, 0, 0)),
            pl.BlockSpec((1, NUM_CLASSES, PP), lambda b: (b, 0, 0)),
        ],
        out_shape=[
            jax.ShapeDtypeStruct((B, 4, PP), jnp.float32),
            jax.ShapeDtypeStruct((B, NUM_CLASSES, PP), jnp.float32),
        ],
    )(al_t, ol_t, pr_t, conf_t, armobj)
    return boxes_t, mconf


def _ds16(off):
    return pl.ds(pl.multiple_of(off, 16), 16)


def _dsh(off, sz):
    return pl.ds(pl.multiple_of(off, 128), sz)


def _sc_body(conf_hbm, boxes_hbm, out_hbm,
             cbuf0, cbuf1, ckey_v, cidx_v, hist_v, skey_v, sidx_v,
             boxes_soa, x1_v, y1_v, x2_v, y2_v, keep_v, outv,
             bsem, csem0, csem1):
    lane = jnp.arange(L, dtype=jnp.int32)
    zi = jnp.zeros((L,), jnp.int32)
    oi = jnp.ones((L,), jnp.int32)
    zf = jnp.zeros((L,), jnp.float32)
    wid = lax.axis_index("s") * NC + lax.axis_index("c")
    bimg = (wid * RPW) // (NUM_CLASSES - 1)
    cbufs = [cbuf0, cbuf1]
    csems = [csem0, csem1]

    bh = [pltpu.async_copy(
        boxes_hbm.at[_dsh((bimg * 4 + c4) * PP, PP)], boxes_soa.at[c4], bsem)
        for c4 in range(4)]

    def ce(a, b):
        ka, va = a
        kb, vb = b
        m = ka >= kb
        return ((jnp.maximum(ka, kb), jnp.where(m, va, vb)),
                (jnp.minimum(ka, kb), jnp.where(m, vb, va)))

    def bmerge(xs):
        if len(xs) == 1:
            k_, v_ = xs[0]
            ks, vs = plsc.sort_key_val(k_, v_, descending=True)
            return [(ks, vs)]
        h = len(xs) // 2
        los, his = [], []
        for i in range(h):
            a, b2 = ce(xs[i], xs[i + h])
            los.append(a)
            his.append(b2)
        return bmerge(los) + bmerge(his)

    def msort(xs):
        if len(xs) == 1:
            k_, v_ = xs[0]
            ks, vs = plsc.sort_key_val(k_, v_, descending=True)
            return [(ks, vs)]
        h = len(xs) // 2
        a = msort(xs[:h])
        b2 = msort(xs[h:])
        b2r = [(lax.rev(k_, (0,)), lax.rev(v_, (0,))) for (k_, v_) in reversed(b2)]
        return bmerge(a + b2r)

    def row_body(w, _):
        r = wid * RPW + w

        rof = (r + bimg + 1) * PP
        h0 = pltpu.async_copy(conf_hbm.at[_dsh(rof, CH)], cbufs[0], csems[0])
        n_v = zi
        handles = [h0, None]
        for ch in range(NCHUNK):
            handles[ch % 2].wait()
            if ch + 1 < NCHUNK:
                handles[(ch + 1) % 2] = pltpu.async_copy(
                    conf_hbm.at[_dsh(rof + (ch + 1) * CH, CH)],
                    cbufs[(ch + 1) % 2], csems[(ch + 1) % 2])

            def comp_body(i, n_v, base=ch * CH, buf=cbufs[ch % 2]):
                c = buf[_ds16(i * L)]
                key = lax.bitcast_convert_type(c, jnp.int32)
                m = c > CONF_THRESH
                mi = jnp.where(m, 1, 0)
                dest = n_v + plsc.cumsum(mi) - 1
                plsc.store_scatter(ckey_v, [dest], key, mask=m)
                plsc.store_scatter(cidx_v, [dest], base + i * L + lane, mask=m)
                return n_v + plsc.all_reduce_population_count(m)

            n_v = plsc.parallel_loop(0, CH // L, carry=n_v, unroll=4)(comp_body)
        n = jnp.sum(jnp.where(lane == 0, n_v, 0))
        nvc = (n + L - 1) // L
        k_t = jnp.minimum(n, TOP_K)

        @pl.when(n > 0)
        def _():
            tb = (nvc - 1) * L
            tm = (tb + lane) >= n
            plsc.store_scatter(ckey_v, [tb + lane], zi, mask=tm)

        pfx = jnp.int32(0)
        rem = k_t
        for li, (sh, psh, wdt) in enumerate(
                [(23, 31, 8), (15, 23, 8), (7, 15, 8), (0, 7, 7)]):
            def z_body(j, hist_v=hist_v):
                hist_v[_ds16(j * L)] = zi

            plsc.parallel_loop(0, SELW, unroll=8)(z_body)

            def h_body(i, sh=sh, psh=psh, pfx=pfx):
                kk = ckey_v[_ds16(i * L)]
                part = lax.shift_right_logical(kk, psh) == pfx
                digit = lax.shift_right_logical(kk, sh) & ((1 << wdt) - 1)
                addr = lane * SELW + digit
                plsc.addupdate_scatter(hist_v, [addr], oi, mask=part)

            plsc.parallel_loop(0, nvc, unroll=2)(h_body)

            def s_body(jj, st):
                carry, rem_c, found, beta = st
                j = 15 - jj
                binv = zi
                for l2 in range(L):
                    binv = binv + hist_v[_ds16(l2 * SELW + j * L)]
                csum = plsc.cumsum(binv)
                tot = jnp.sum(binv)
                suf = tot - csum + binv + carry
                fm = suf >= rem_c
                mc = jnp.sum(jnp.where(fm, 1, 0))
                here = (found == 0) & (mc > 0)
                bl = mc - 1
                above = jnp.sum(jnp.where(lane == bl, suf - binv, 0))
                beta = jnp.where(here, j * L + bl, beta)
                rem_c = jnp.where(here, rem_c - above, rem_c)
                found = jnp.where(here, jnp.int32(1), found)
                carry = carry + tot
                return (carry, rem_c, found, beta)

            _, rem, _, beta = lax.fori_loop(
                0, L, s_body, (jnp.int32(0), rem, jnp.int32(0), jnp.int32(0)))
            pfx = beta if li == 0 else ((pfx << wdt) | beta)
        t_thr = pfx

        for t in range(SELW // L):
            skey_v[_ds16(t * L)] = zi
            sidx_v[_ds16(t * L)] = zi

        def sel_body(i, st):
            cnt_v, eqc_v = st
            kk = ckey_v[_ds16(i * L)]
            mgt = kk > t_thr
            meq = kk == t_thr
            meqi = jnp.where(meq, 1, 0)
            eqr = eqc_v + plsc.cumsum(meqi)
            msel = mgt | (meq & (eqr <= rem))
            mseli = jnp.where(msel, 1, 0)
            dest = cnt_v + plsc.cumsum(mseli) - 1
            plsc.store_scatter(skey_v, [dest], kk, mask=msel)
            plsc.store_scatter(sidx_v, [dest], cidx_v[_ds16(i * L)], mask=msel)
            return (cnt_v + plsc.all_reduce_population_count(msel),
                    eqc_v + plsc.all_reduce_population_count(meq))

        plsc.parallel_loop(0, nvc, carry=(zi, zi), unroll=2)(sel_body)

        pairs = [(skey_v[_ds16(t * L)], sidx_v[_ds16(t * L)])
                 for t in range(SELW // L)]
        pairs = msort(pairs)
        ks = [p[0] for p in pairs]
        vs = [p[1] for p in pairs]
        rotl = jnp.minimum(lane + 1, 15)
        rotr = jnp.maximum(lane - 1, 0)
        xor1 = lane ^ 1
        even_lane = (lane & 1) == 0
        for _ in range(2):
            for t in range(16):
                pk = ks[t][xor1]
                pv = vs[t][xor1]
                eq = ks[t] == pk
                take = eq & jnp.where(even_lane, pv < vs[t], pv > vs[t])
                vs[t] = jnp.where(take, pv, vs[t])
            nks, nvs = [], []
            for t in range(16):
                nk = ks[t][rotl]
                nv = vs[t][rotl]
                if t < 15:
                    s0k = jnp.sum(jnp.where(lane == 0, ks[t + 1], 0))
                    s0v = jnp.sum(jnp.where(lane == 0, vs[t + 1], 0))
                else:
                    s0k, s0v = jnp.int32(-1), jnp.int32(0)
                nks.append(jnp.where(lane == 15, s0k, nk))
                nvs.append(jnp.where(lane == 15, s0v, nv))
            pks, pvs = [], []
            for t in range(16):
                pk = ks[t][rotr]
                pv = vs[t][rotr]
                if t > 0:
                    s15k = jnp.sum(jnp.where(lane == 15, ks[t - 1], 0))
                    s15v = jnp.sum(jnp.where(lane == 15, vs[t - 1], 0))
                else:
                    s15k, s15v = jnp.int32(-1), jnp.int32(0)
                pks.append(jnp.where(lane == 0, s15k, pk))
                pvs.append(jnp.where(lane == 0, s15v, pv))
            for t in range(16):
                eq_n = ks[t] == nks[t]
                eq_p = ks[t] == pks[t]
                take_n = (~even_lane) & eq_n & (nvs[t] < vs[t])
                take_p = even_lane & eq_p & (pvs[t] > vs[t])
                vs[t] = jnp.where(take_n, nvs[t],
                                  jnp.where(take_p, pvs[t], vs[t]))
        for t in range(NBV):
            skey_v[_ds16(t * L)] = ks[t]

        @pl.when(w == 0)
        def _():
            for h in bh:
                h.wait()
        for t in range(NBV):
            rl = t * L + lane
            bi = vs[t]
            x1 = plsc.load_gather(boxes_soa, [zi, bi])
            y1 = plsc.load_gather(boxes_soa, [zi + 1, bi])
            x2 = plsc.load_gather(boxes_soa, [zi + 2, bi])
            y2 = plsc.load_gather(boxes_soa, [zi + 3, bi])
            x1_v[_ds16(t * L)] = x1
            y1_v[_ds16(t * L)] = y1
            x2_v[_ds16(t * L)] = x2
            y2_v[_ds16(t * L)] = y2
            keep_v[_ds16(t * L)] = jnp.where(rl < k_t, 1, 0)

        def nms_body(i, _):
            tv = i // L
            off = tv * L
            liv = (i - off) + zi
            kmask = keep_v[_ds16(off)][liv] > 0
            bx1 = x1_v[_ds16(off)][liv]
            by1 = y1_v[_ds16(off)][liv]
            bx2 = x2_v[_ds16(off)][liv]
            by2 = y2_v[_ds16(off)][liv]
            bar = (bx2 - bx1) * (by2 - by1)

            def sup(o2, first):
                jx1 = x1_v[_ds16(o2)]
                jy1 = y1_v[_ds16(o2)]
                jx2 = x2_v[_ds16(o2)]
                jy2 = y2_v[_ds16(o2)]
                xx1 = jnp.maximum(jx1, bx1)
                yy1 = jnp.maximum(jy1, by1)
                xx2 = jnp.minimum(jx2, bx2)
                yy2 = jnp.minimum(jy2, by2)
                iw = jnp.maximum(xx2 - xx1, 0.0)
                ih = jnp.maximum(yy2 - yy1, 0.0)
                inter = iw * ih
                union = (jx2 - jx1) * (jy2 - jy1) - inter + bar
                s = (inter > NMS_THRESH * union) & kmask
                if first:
                    s = s & ((o2 + lane) > i)
                kv = keep_v[_ds16(o2)]
                keep_v[_ds16(o2)] = jnp.where(s, 0, kv)

            sup(off, True)

            def sup_body(t2):
                sup(t2 * L, False)

            plsc.parallel_loop(tv + 1, NBV, unroll=4)(sup_body)
            return 0

        lax.fori_loop(0, k_t, nms_body, 0)

        for t in range(OW // L):
            outv[_ds16(t * L)] = zf
        cnt_v = zi
        for t in range(NBV):
            kv = keep_v[_ds16(t * L)]
            m = kv > 0
            mi = jnp.where(m, 1, 0)
            dest = cnt_v + plsc.cumsum(mi) - 1
            sc = lax.bitcast_convert_type(skey_v[_ds16(t * L)], jnp.float32)
            plsc.store_scatter(outv, [dest], sc, mask=m)
            plsc.store_scatter(outv, [dest + SELW], x1_v[_ds16(t * L)], mask=m)
            plsc.store_scatter(outv, [dest + 2 * SELW], y1_v[_ds16(t * L)], mask=m)
            plsc.store_scatter(outv, [dest + 3 * SELW], x2_v[_ds16(t * L)], mask=m)
            plsc.store_scatter(outv, [dest + 4 * SELW], y2_v[_ds16(t * L)], mask=m)
            cnt_v = cnt_v + plsc.all_reduce_population_count(m)
        pltpu.sync_copy(outv, out_hbm.at[_dsh(r * OW, OW)])
        return 0

    lax.fori_loop(0, RPW, row_body, 0)


def _sc_nms(conf_rows, boxes_flat):
    mesh = plsc.VectorSubcoreMesh(core_axis_name="c", subcore_axis_name="s")
    f = functools.partial(
        pl.kernel,
        mesh=mesh,
        compiler_params=pltpu.CompilerParams(needs_layout_passes=False),
        out_type=jax.ShapeDtypeStruct((ROWS * OW,), jnp.float32),
        scratch_types=[
            pltpu.VMEM((CH,), jnp.float32),
            pltpu.VMEM((CH,), jnp.float32),
            pltpu.VMEM((P,), jnp.int32),
            pltpu.VMEM((P,), jnp.int32),
            pltpu.VMEM((L * SELW,), jnp.int32),
            pltpu.VMEM((SELW,), jnp.int32),
            pltpu.VMEM((SELW,), jnp.int32),
            pltpu.VMEM((4, PP), jnp.float32),
            pltpu.VMEM((SELW,), jnp.float32),
            pltpu.VMEM((SELW,), jnp.float32),
            pltpu.VMEM((SELW,), jnp.float32),
            pltpu.VMEM((SELW,), jnp.float32),
            pltpu.VMEM((SELW,), jnp.int32),
            pltpu.VMEM((OW,), jnp.float32),
            pltpu.SemaphoreType.DMA,
            pltpu.SemaphoreType.DMA,
            pltpu.SemaphoreType.DMA,
        ],
    )(_sc_body)
    return f(conf_rows, boxes_flat)


def kernel(arm_loc_data, arm_conf_data, odm_loc_data, odm_conf_data, prior_data):
    conf_t = jnp.transpose(odm_conf_data, (0, 2, 1))
    armobj = arm_conf_data[:, :, 1][:, None, :]
    boxes, mconf = _decode_boxes(
        arm_loc_data, odm_loc_data, prior_data, conf_t, armobj)
    out = _sc_nms(mconf.reshape(B * NUM_CLASSES * PP), boxes.reshape(B * 4 * PP))
    out = out.reshape(ROWS, 5, SELW)
    out = jnp.transpose(out, (0, 2, 1))[:, :TOP_K, :].reshape(
        B, NUM_CLASSES - 1, TOP_K, 5)
    bg = jnp.zeros((B, 1, TOP_K, 5), jnp.float32)
    return jnp.concatenate([bg, out], axis=1)


# speedup vs baseline: 1.0414x; 1.0414x over previous
import functools

import jax
import jax.numpy as jnp
from jax import lax
from jax.experimental import pallas as pl
from jax.experimental.pallas import tpu as pltpu
from jax.experimental.pallas import tpu_sc as plsc

NUM_CLASSES = 21
TOP_K = 200
CONF_THRESH = 0.01
NMS_THRESH = 0.45
OBJ_THRESH = 0.01
V0, V1 = 0.1, 0.2
B, P = 8, 16320

L = 16
ROWS = B * (NUM_CLASSES - 1)
NC, NS = 2, 16
NW = NC * NS
RPW = ROWS // NW
SELW = 256
NBV = 13
PP = 16384
NCHUNK = 4
CH = PP // NCHUNK
OW = 5 * SELW


def _decode_body(al_ref, ol_ref, pr_ref, cf_ref, ao_ref, out_ref, mc_ref):
    al = al_ref[0]
    ol = ol_ref[0]
    pr = pr_ref[...]
    pcx, pcy, pw, ph = pr[0:1], pr[1:2], pr[2:3], pr[3:4]
    dcx = pcx + al[0:1] * V0 * pw
    dcy = pcy + al[1:2] * V0 * ph
    dw = pw * jnp.exp(al[2:3] * V1)
    dh = ph * jnp.exp(al[3:4] * V1)
    x1 = dcx - dw / 2.0
    y1 = dcy - dh / 2.0
    x2 = dcx + dw / 2.0
    y2 = dcy + dh / 2.0
    dcx = (x2 + x1) / 2.0
    dcy = (y2 + y1) / 2.0
    dw = x2 - x1
    dh = y2 - y1
    bcx = dcx + ol[0:1] * V0 * dw
    bcy = dcy + ol[1:2] * V0 * dh
    bw = dw * jnp.exp(ol[2:3] * V1)
    bh = dh * jnp.exp(ol[3:4] * V1)
    zpad4 = jnp.zeros((4, PP - P), jnp.float32)
    zpad21 = jnp.zeros((NUM_CLASSES, PP - P), jnp.float32)
    box4 = jnp.concatenate(
        [bcx - bw / 2.0, bcy - bh / 2.0, bcx + bw / 2.0, bcy + bh / 2.0], axis=0)
    out_ref[0] = jnp.concatenate([box4, zpad4], axis=1)
    mc = jnp.where(ao_ref[0] > OBJ_THRESH, cf_ref[0], 0.0)
    mc_ref[0] = jnp.concatenate([mc, zpad21], axis=1)


def _decode_boxes(arm_loc, odm_loc, priors, conf_t, armobj):
    al_t = jnp.transpose(arm_loc, (0, 2, 1))
    ol_t = jnp.transpose(odm_loc, (0, 2, 1))
    pr_t = jnp.transpose(priors, (1, 0))
    boxes_t, mconf = pl.pallas_call(
        _decode_body,
        grid=(B,),
        in_specs=[
            pl.BlockSpec((1, 4, P), lambda b: (b, 0, 0)),
            pl.BlockSpec((1, 4, P), lambda b: (b, 0, 0)),
            pl.BlockSpec((4, P), lambda b: (0, 0)),
            pl.BlockSpec((1, NUM_CLASSES, P), lambda b: (b, 0, 0)),
            pl.BlockSpec((1, 1, P), lambda b: (b, 0, 0)),
        ],
        out_specs=[
            pl.BlockSpec((1, 4, PP), lambda b: (b, 0, 0)),
            pl.BlockSpec((1, NUM_CLASSES, PP), lambda b: (b, 0, 0)),
        ],
        out_shape=[
            jax.ShapeDtypeStruct((B, 4, PP), jnp.float32),
            jax.ShapeDtypeStruct((B, NUM_CLASSES, PP), jnp.float32),
        ],
    )(al_t, ol_t, pr_t, conf_t, armobj)
    return boxes_t, mconf


def _ds16(off):
    return pl.ds(pl.multiple_of(off, 16), 16)


def _dsh(off, sz):
    return pl.ds(pl.multiple_of(off, 128), sz)


def _sc_body(conf_hbm, boxes_hbm, out_hbm,
             cbuf0, cbuf1, ckey_v, cidx_v, hist_v, skey_v, sidx_v,
             boxes_soa, x1_v, y1_v, x2_v, y2_v, keep_v, outv,
             bsem, csem0, csem1):
    lane = jnp.arange(L, dtype=jnp.int32)
    zi = jnp.zeros((L,), jnp.int32)
    oi = jnp.ones((L,), jnp.int32)
    zf = jnp.zeros((L,), jnp.float32)
    wid = lax.axis_index("s") * NC + lax.axis_index("c")
    bimg = (wid * RPW) // (NUM_CLASSES - 1)
    cbufs = [cbuf0, cbuf1]
    csems = [csem0, csem1]

    bh = [pltpu.async_copy(
        boxes_hbm.at[_dsh((bimg * 4 + c4) * PP, PP)], boxes_soa.at[c4], bsem)
        for c4 in range(4)]

    def ce(a, b):
        ka, va = a
        kb, vb = b
        m = ka >= kb
        return ((jnp.maximum(ka, kb), jnp.where(m, va, vb)),
                (jnp.minimum(ka, kb), jnp.where(m, vb, va)))

    def bmerge(xs):
        if len(xs) == 1:
            k_, v_ = xs[0]
            ks, vs = plsc.sort_key_val(k_, v_, descending=True)
            return [(ks, vs)]
        h = len(xs) // 2
        los, his = [], []
        for i in range(h):
            a, b2 = ce(xs[i], xs[i + h])
            los.append(a)
            his.append(b2)
        return bmerge(los) + bmerge(his)

    def msort(xs):
        if len(xs) == 1:
            k_, v_ = xs[0]
            ks, vs = plsc.sort_key_val(k_, v_, descending=True)
            return [(ks, vs)]
        h = len(xs) // 2
        a = msort(xs[:h])
        b2 = msort(xs[h:])
        b2r = [(lax.rev(k_, (0,)), lax.rev(v_, (0,))) for (k_, v_) in reversed(b2)]
        return bmerge(a + b2r)

    def row_body(w, _):
        r = wid * RPW + w

        rof = (r + bimg + 1) * PP
        h0 = pltpu.async_copy(conf_hbm.at[_dsh(rof, CH)], cbufs[0], csems[0])
        n_v = zi
        handles = [h0, None]
        for ch in range(NCHUNK):
            handles[ch % 2].wait()
            if ch + 1 < NCHUNK:
                handles[(ch + 1) % 2] = pltpu.async_copy(
                    conf_hbm.at[_dsh(rof + (ch + 1) * CH, CH)],
                    cbufs[(ch + 1) % 2], csems[(ch + 1) % 2])

            def comp_body(i, n_v, base=ch * CH, buf=cbufs[ch % 2]):
                c = buf[_ds16(i * L)]
                key = lax.bitcast_convert_type(c, jnp.int32)
                m = c > CONF_THRESH
                mi = jnp.where(m, 1, 0)
                dest = n_v + plsc.cumsum(mi) - 1
                plsc.store_scatter(ckey_v, [dest], key, mask=m)
                plsc.store_scatter(cidx_v, [dest], base + i * L + lane, mask=m)
                return n_v + plsc.all_reduce_population_count(m)

            n_v = plsc.parallel_loop(0, CH // L, carry=n_v, unroll=4)(comp_body)
        n = jnp.sum(jnp.where(lane == 0, n_v, 0))
        nvc = (n + L - 1) // L
        k_t = jnp.minimum(n, TOP_K)

        @pl.when(n > 0)
        def _():
            tb = (nvc - 1) * L
            tm = (tb + lane) >= n
            plsc.store_scatter(ckey_v, [tb + lane], zi, mask=tm)

        pfx = jnp.int32(0)
        rem = k_t
        for li, (sh, psh, wdt) in enumerate(
                [(23, 31, 8), (15, 23, 8), (7, 15, 8), (0, 7, 7)]):
            def z_body(j, hist_v=hist_v):
                hist_v[_ds16(j * L)] = zi

            plsc.parallel_loop(0, SELW, unroll=8)(z_body)

            def h_body(i, sh=sh, psh=psh, pfx=pfx):
                kk = ckey_v[_ds16(i * L)]
                part = lax.shift_right_logical(kk, psh) == pfx
                digit = lax.shift_right_logical(kk, sh) & ((1 << wdt) - 1)
                addr = lane * SELW + digit
                plsc.addupdate_scatter(hist_v, [addr], oi, mask=part)

            plsc.parallel_loop(0, nvc, unroll=2)(h_body)

            def s_body(jj, st):
                carry, rem_c, found, beta = st
                j = 15 - jj
                binv = zi
                for l2 in range(L):
                    binv = binv + hist_v[_ds16(l2 * SELW + j * L)]
                csum = plsc.cumsum(binv)
                tot = jnp.sum(binv)
                suf = tot - csum + binv + carry
                fm = suf >= rem_c
                mc = jnp.sum(jnp.where(fm, 1, 0))
                here = (found == 0) & (mc > 0)
                bl = mc - 1
                above = jnp.sum(jnp.where(lane == bl, suf - binv, 0))
                beta = jnp.where(here, j * L + bl, beta)
                rem_c = jnp.where(here, rem_c - above, rem_c)
                found = jnp.where(here, jnp.int32(1), found)
                carry = carry + tot
                return (carry, rem_c, found, beta)

            _, rem, _, beta = lax.fori_loop(
                0, L, s_body, (jnp.int32(0), rem, jnp.int32(0), jnp.int32(0)))
            pfx = beta if li == 0 else ((pfx << wdt) | beta)
        t_thr = pfx

        for t in range(SELW // L):
            skey_v[_ds16(t * L)] = zi
            sidx_v[_ds16(t * L)] = zi

        def sel_body(i, st):
            cnt_v, eqc_v = st
            kk = ckey_v[_ds16(i * L)]
            mgt = kk > t_thr
            meq = kk == t_thr
            meqi = jnp.where(meq, 1, 0)
            eqr = eqc_v + plsc.cumsum(meqi)
            msel = mgt | (meq & (eqr <= rem))
            mseli = jnp.where(msel, 1, 0)
            dest = cnt_v + plsc.cumsum(mseli) - 1
            plsc.store_scatter(skey_v, [dest], kk, mask=msel)
            plsc.store_scatter(sidx_v, [dest], cidx_v[_ds16(i * L)], mask=msel)
            return (cnt_v + plsc.all_reduce_population_count(msel),
                    eqc_v + plsc.all_reduce_population_count(meq))

        plsc.parallel_loop(0, nvc, carry=(zi, zi), unroll=2)(sel_body)

        pairs = [(skey_v[_ds16(t * L)], sidx_v[_ds16(t * L)])
                 for t in range(SELW // L)]
        pairs = msort(pairs)
        ks = [p[0] for p in pairs]
        vs = [p[1] for p in pairs]
        rotl = jnp.minimum(lane + 1, 15)
        rotr = jnp.maximum(lane - 1, 0)
        xor1 = lane ^ 1
        even_lane = (lane & 1) == 0
        for _ in range(2):
            for t in range(16):
                pk = ks[t][xor1]
                pv = vs[t][xor1]
                eq = ks[t] == pk
                take = eq & jnp.where(even_lane, pv < vs[t], pv > vs[t])
                vs[t] = jnp.where(take, pv, vs[t])
            nks, nvs = [], []
            for t in range(16):
                nk = ks[t][rotl]
                nv = vs[t][rotl]
                if t < 15:
                    s0k = jnp.sum(jnp.where(lane == 0, ks[t + 1], 0))
                    s0v = jnp.sum(jnp.where(lane == 0, vs[t + 1], 0))
                else:
                    s0k, s0v = jnp.int32(-1), jnp.int32(0)
                nks.append(jnp.where(lane == 15, s0k, nk))
                nvs.append(jnp.where(lane == 15, s0v, nv))
            pks, pvs = [], []
            for t in range(16):
                pk = ks[t][rotr]
                pv = vs[t][rotr]
                if t > 0:
                    s15k = jnp.sum(jnp.where(lane == 15, ks[t - 1], 0))
                    s15v = jnp.sum(jnp.where(lane == 15, vs[t - 1], 0))
                else:
                    s15k, s15v = jnp.int32(-1), jnp.int32(0)
                pks.append(jnp.where(lane == 0, s15k, pk))
                pvs.append(jnp.where(lane == 0, s15v, pv))
            for t in range(16):
                eq_n = ks[t] == nks[t]
                eq_p = ks[t] == pks[t]
                take_n = (~even_lane) & eq_n & (nvs[t] < vs[t])
                take_p = even_lane & eq_p & (pvs[t] > vs[t])
                vs[t] = jnp.where(take_n, nvs[t],
                                  jnp.where(take_p, pvs[t], vs[t]))
        for t in range(NBV):
            skey_v[_ds16(t * L)] = ks[t]

        @pl.when(w == 0)
        def _():
            for h in bh:
                h.wait()
        for t in range(NBV):
            rl = t * L + lane
            bi = vs[t]
            x1 = plsc.load_gather(boxes_soa, [zi, bi])
            y1 = plsc.load_gather(boxes_soa, [zi + 1, bi])
            x2 = plsc.load_gather(boxes_soa, [zi + 2, bi])
            y2 = plsc.load_gather(boxes_soa, [zi + 3, bi])
            x1_v[_ds16(t * L)] = x1
            y1_v[_ds16(t * L)] = y1
            x2_v[_ds16(t * L)] = x2
            y2_v[_ds16(t * L)] = y2
            keep_v[_ds16(t * L)] = jnp.where(rl < k_t, 1, 0)

        def nms_body(i, _):
            tv = i // L
            off = tv * L
            li_ = i - off
            kvec = keep_v[_ds16(off)]
            ki = jnp.sum(jnp.where(lane == li_, kvec, 0))

            @pl.when(ki > 0)
            def _():
                liv = li_ + zi
                bx1 = x1_v[_ds16(off)][liv]
                by1 = y1_v[_ds16(off)][liv]
                bx2 = x2_v[_ds16(off)][liv]
                by2 = y2_v[_ds16(off)][liv]
                bar = (bx2 - bx1) * (by2 - by1)

                def sup(o2, first):
                    jx1 = x1_v[_ds16(o2)]
                    jy1 = y1_v[_ds16(o2)]
                    jx2 = x2_v[_ds16(o2)]
                    jy2 = y2_v[_ds16(o2)]
                    xx1 = jnp.maximum(jx1, bx1)
                    yy1 = jnp.maximum(jy1, by1)
                    xx2 = jnp.minimum(jx2, bx2)
                    yy2 = jnp.minimum(jy2, by2)
                    iw = jnp.maximum(xx2 - xx1, 0.0)
                    ih = jnp.maximum(yy2 - yy1, 0.0)
                    inter = iw * ih
                    union = (jx2 - jx1) * (jy2 - jy1) - inter + bar
                    s = inter > NMS_THRESH * union
                    if first:
                        s = s & ((o2 + lane) > i)
                    kv = keep_v[_ds16(o2)]
                    keep_v[_ds16(o2)] = jnp.where(s, 0, kv)

                sup(off, True)

                def sup_body(t2):
                    sup(t2 * L, False)

                plsc.parallel_loop(tv + 1, NBV, unroll=2)(sup_body)

            return 0

        lax.fori_loop(0, k_t, nms_body, 0)

        for t in range(OW // L):
            outv[_ds16(t * L)] = zf
        cnt_v = zi
        for t in range(NBV):
            kv = keep_v[_ds16(t * L)]
            m = kv > 0
            mi = jnp.where(m, 1, 0)
            dest = cnt_v + plsc.cumsum(mi) - 1
            sc = lax.bitcast_convert_type(skey_v[_ds16(t * L)], jnp.float32)
            plsc.store_scatter(outv, [dest], sc, mask=m)
            plsc.store_scatter(outv, [dest + SELW], x1_v[_ds16(t * L)], mask=m)
            plsc.store_scatter(outv, [dest + 2 * SELW], y1_v[_ds16(t * L)], mask=m)
            plsc.store_scatter(outv, [dest + 3 * SELW], x2_v[_ds16(t * L)], mask=m)
            plsc.store_scatter(outv, [dest + 4 * SELW], y2_v[_ds16(t * L)], mask=m)
            cnt_v = cnt_v + plsc.all_reduce_population_count(m)
        pltpu.sync_copy(outv, out_hbm.at[_dsh(r * OW, OW)])
        return 0

    lax.fori_loop(0, RPW, row_body, 0)


def _sc_nms(conf_rows, boxes_flat):
    mesh = plsc.VectorSubcoreMesh(core_axis_name="c", subcore_axis_name="s")
    f = functools.partial(
        pl.kernel,
        mesh=mesh,
        compiler_params=pltpu.CompilerParams(needs_layout_passes=False),
        out_type=jax.ShapeDtypeStruct((ROWS * OW,), jnp.float32),
        scratch_types=[
            pltpu.VMEM((CH,), jnp.float32),
            pltpu.VMEM((CH,), jnp.float32),
            pltpu.VMEM((P,), jnp.int32),
            pltpu.VMEM((P,), jnp.int32),
            pltpu.VMEM((L * SELW,), jnp.int32),
            pltpu.VMEM((SELW,), jnp.int32),
            pltpu.VMEM((SELW,), jnp.int32),
            pltpu.VMEM((4, PP), jnp.float32),
            pltpu.VMEM((SELW,), jnp.float32),
            pltpu.VMEM((SELW,), jnp.float32),
            pltpu.VMEM((SELW,), jnp.float32),
            pltpu.VMEM((SELW,), jnp.float32),
            pltpu.VMEM((SELW,), jnp.int32),
            pltpu.VMEM((OW,), jnp.float32),
            pltpu.SemaphoreType.DMA,
            pltpu.SemaphoreType.DMA,
            pltpu.SemaphoreType.DMA,
        ],
    )(_sc_body)
    return f(conf_rows, boxes_flat)


def kernel(arm_loc_data, arm_conf_data, odm_loc_data, odm_conf_data, prior_data):
    conf_t = jnp.transpose(odm_conf_data, (0, 2, 1))
    armobj = arm_conf_data[:, :, 1][:, None, :]
    boxes, mconf = _decode_boxes(
        arm_loc_data, odm_loc_data, prior_data, conf_t, armobj)
    out = _sc_nms(mconf.reshape(B * NUM_CLASSES * PP), boxes.reshape(B * 4 * PP))
    out = out.reshape(ROWS, 5, SELW)
    out = jnp.transpose(out, (0, 2, 1))[:, :TOP_K, :].reshape(
        B, NUM_CLASSES - 1, TOP_K, 5)
    bg = jnp.zeros((B, 1, TOP_K, 5), jnp.float32)
    return jnp.concatenate([bg, out], axis=1)
